# baseline (speedup 1.0000x reference)
import functools

import jax
import jax.numpy as jnp
from jax import lax
from jax.experimental import pallas as pl
from jax.experimental.pallas import tpu as pltpu
from jax.experimental.pallas import tpu_sc as plsc

_BATCH = 4096
_HIST = 50
_D = 64
_DP = 128
_B = _BATCH * _HIST
_NV = 1000000
_NW = 32
_B_PER_W = _B // _NW
_BLANES = 128
_CH_H = 2
_CH_ROWS = _CH_H * _BLANES
_NCH = _HIST // _CH_H
_NJ = _HIST * (_D // 8) * (_BATCH // _BLANES)

_TCW = 7936
_TC_GRID = 126
_NMAIN = _TCW * _TC_GRID


def _repack_body(tt_hbm, tail_hbm, out_hbm, vin, vout, vtin, vtout, sin, sout, stail):
    g = pl.program_id(0)
    s = lax.rem(g, 2)

    def in_copy(blk, slot):
        return pltpu.make_async_copy(
            tt_hbm.at[:, pl.ds(blk * _TCW, _TCW)], vin.at[slot], sin.at[slot]
        )

    def out_copy(blk, slot):
        return pltpu.make_async_copy(
            vout.at[slot], out_hbm.at[pl.ds(blk * _TCW, _TCW)], sout.at[slot]
        )

    @pl.when(g == 0)
    def _():
        in_copy(0, 0).start()
        in_copy(1, 1).start()

    in_copy(g, s).wait()
    t = jnp.transpose(vin[s], (1, 0))
    vout[s] = jnp.concatenate([t, jnp.zeros((_TCW, _D), jnp.float32)], axis=1)

    @pl.when(g >= 2)
    def _():
        out_copy(g - 2, s).wait()

    out_copy(g, s).start()

    @pl.when(g + 2 < _TC_GRID)
    def _():
        in_copy(g + 2, s).start()

    @pl.when(g == _TC_GRID - 1)
    def _():
        tin = pltpu.make_async_copy(tail_hbm, vtin, stail.at[0])
        tin.start()
        tin.wait()
        vtout[...] = jnp.concatenate(
            [vtin[...], jnp.zeros((_NV - _NMAIN, _D), jnp.float32)], axis=1
        )
        tout = pltpu.make_async_copy(
            vtout, out_hbm.at[pl.ds(_NMAIN, _NV - _NMAIN)], stail.at[1]
        )
        tout.start()
        out_copy(g - 1, lax.rem(g + 1, 2)).wait()
        out_copy(g, s).wait()
        tout.wait()


@jax.jit
def _tc_repack(tt, tail):
    return pl.pallas_call(
        _repack_body,
        grid=(_TC_GRID,),
        in_specs=[
            pl.BlockSpec(memory_space=pl.ANY),
            pl.BlockSpec(memory_space=pl.ANY),
        ],
        out_specs=pl.BlockSpec(memory_space=pl.ANY),
        out_shape=jax.ShapeDtypeStruct((_NV, _DP), jnp.float32),
        scratch_shapes=[
            pltpu.VMEM((2, _D, _TCW), jnp.float32),
            pltpu.VMEM((2, _TCW, _DP), jnp.float32),
            pltpu.VMEM((_NV - _NMAIN, _D), jnp.float32),
            pltpu.VMEM((_NV - _NMAIN, _DP), jnp.float32),
            pltpu.SemaphoreType.DMA((2,)),
            pltpu.SemaphoreType.DMA((2,)),
            pltpu.SemaphoreType.DMA((2,)),
        ],
    )(tt, tail)


def _gather_body(
    idx_hbm, table_hbm, out_hbm, idx_v, idx2, g0, g1, t0, t1, sg0, sg1, st0, st1
):
    wid = lax.axis_index("s") * 2 + lax.axis_index("c")
    base = wid * _B_PER_W
    pltpu.sync_copy(idx_hbm.at[pl.ds(base, _B_PER_W)], idx_v)

    iota = lax.iota(jnp.int32, 16)

    def reorder(g, _):
        h = g // 8
        blgrp = lax.rem(g, 8)
        srcv = (blgrp * 16 + iota) * _HIST + h
        idx2[pl.ds(g * 16, 16)] = plsc.load_gather(idx_v, [srcv])
        return ()

    lax.fori_loop(0, _B_PER_W // 16, reorder, (), unroll=False)

    def start_gather(c, gbuf, sem):
        return pltpu.async_copy(
            table_hbm.at[idx2.at[pl.ds(c * _CH_ROWS, _CH_ROWS)]], gbuf, sem
        )

    def process(c, gbuf, tbuf, tsem):
        def transpose_k(k, _):
            h2 = k // 64
            tr = lax.rem(k // 8, 8)
            sb = lax.rem(k, 8)
            col = tr * 8 + sb
            colv = jnp.broadcast_to(col, (16,))
            row0 = h2 * _BLANES
            for lg in range(8):
                rows = row0 + lg * 16 + iota
                v = plsc.load_gather(gbuf, [rows, colv])
                tbuf[pl.ds(k * 128 + lg * 16, 16)] = v
            return ()

        lax.fori_loop(0, 2 * 8 * 8, transpose_k, (), unroll=False)

        copies = []
        for m in range(16):
            h2, tr = m // 8, m % 8
            j = ((c * _CH_H + h2) * 8 + tr) * _NW + wid
            copies.append(
                pltpu.async_copy(
                    tbuf.at[pl.ds(m * 1024, 1024)], out_hbm.at[j], tsem
                )
            )
        for cp in copies:
            cp.wait()

    cp0 = start_gather(0, g0, sg0)

    def body2(i, _):
        c0 = i * 2
        pltpu.async_copy(
            table_hbm.at[idx2.at[pl.ds((c0 + 1) * _CH_ROWS, _CH_ROWS)]], g1, sg1
        )
        pltpu.make_async_copy(
            table_hbm.at[idx2.at[pl.ds(c0 * _CH_ROWS, _CH_ROWS)]], g0, sg0
        ).wait()
        process(c0, g0, t0, st0)
        pltpu.async_copy(
            table_hbm.at[idx2.at[pl.ds((c0 + 2) * _CH_ROWS, _CH_ROWS)]], g0, sg0
        )
        pltpu.make_async_copy(
            table_hbm.at[idx2.at[pl.ds((c0 + 1) * _CH_ROWS, _CH_ROWS)]], g1, sg1
        ).wait()
        process(c0 + 1, g1, t1, st1)
        return ()

    lax.fori_loop(0, (_NCH - 1) // 2, body2, (), unroll=False)
    cp_last = pltpu.make_async_copy(
        table_hbm.at[idx2.at[pl.ds((_NCH - 1) * _CH_ROWS, _CH_ROWS)]], g0, sg0
    )
    cp_last.wait()
    process(_NCH - 1, g0, t0, st0)
    del cp0


@jax.jit
def _sc_gather(idx_flat, table128):
    mesh = plsc.VectorSubcoreMesh(core_axis_name="c", subcore_axis_name="s")
    fn = functools.partial(
        pl.kernel,
        mesh=mesh,
        out_type=jax.ShapeDtypeStruct((_NJ, 8 * _BLANES), jnp.float32),
        scratch_types=[
            pltpu.VMEM((_B_PER_W,), jnp.int32),
            pltpu.VMEM((_B_PER_W,), jnp.int32),
            pltpu.VMEM((_CH_ROWS, _DP), jnp.float32),
            pltpu.VMEM((_CH_ROWS, _DP), jnp.float32),
            pltpu.VMEM((16 * 8 * _BLANES,), jnp.float32),
            pltpu.VMEM((16 * 8 * _BLANES,), jnp.float32),
            pltpu.SemaphoreType.DMA,
            pltpu.SemaphoreType.DMA,
            pltpu.SemaphoreType.DMA,
            pltpu.SemaphoreType.DMA,
        ],
        compiler_params=pltpu.CompilerParams(
            use_tc_tiling_on_sc=True, needs_layout_passes=False
        ),
    )(_gather_body)
    return fn(idx_flat, table128)


def kernel(inputs, entity_table, relation_table):
    idx_flat = inputs.reshape(_B).astype(jnp.int32)
    table128 = _tc_repack(entity_table.T, entity_table[_NMAIN:])
    out_t = _sc_gather(idx_flat, table128)
    o5 = out_t.reshape(_HIST, _D // 8, _BATCH // _BLANES, 8, _BLANES)
    return o5.transpose(2, 4, 0, 1, 3).reshape(_BATCH, _HIST, _D)


# speedup vs baseline: 1.2694x; 1.2694x over previous
import functools

import jax
import jax.numpy as jnp
from jax import lax
from jax.experimental import pallas as pl
from jax.experimental.pallas import tpu as pltpu
from jax.experimental.pallas import tpu_sc as plsc

_BATCH = 4096
_HIST = 50
_D = 64
_DP = 128
_B = _BATCH * _HIST
_NV = 1000000
_NW = 32
_B_PER_W = _B // _NW
_CHUNK = 400
_NCHUNK = _B_PER_W // _CHUNK

_TCW = 7936
_TC_GRID = 126
_NMAIN = _TCW * _TC_GRID


def _repack_body(tt_hbm, tail_hbm, out_hbm, vin, vout, vtin, vtout, sin, sout, stail):
    g = pl.program_id(0)
    s = lax.rem(g, 2)

    def in_copy(blk, slot):
        return pltpu.make_async_copy(
            tt_hbm.at[:, pl.ds(blk * _TCW, _TCW)], vin.at[slot], sin.at[slot]
        )

    def out_copy(blk, slot):
        return pltpu.make_async_copy(
            vout.at[slot], out_hbm.at[pl.ds(blk * _TCW, _TCW)], sout.at[slot]
        )

    @pl.when(g == 0)
    def _():
        in_copy(0, 0).start()
        in_copy(1, 1).start()

    in_copy(g, s).wait()
    t = jnp.transpose(vin[s], (1, 0))
    vout[s] = jnp.concatenate([t, jnp.zeros((_TCW, _D), jnp.float32)], axis=1)

    @pl.when(g >= 2)
    def _():
        out_copy(g - 2, s).wait()

    out_copy(g, s).start()

    @pl.when(g + 2 < _TC_GRID)
    def _():
        in_copy(g + 2, s).start()

    @pl.when(g == _TC_GRID - 1)
    def _():
        tin = pltpu.make_async_copy(tail_hbm, vtin, stail.at[0])
        tin.start()
        tin.wait()
        vtout[...] = jnp.concatenate(
            [vtin[...], jnp.zeros((_NV - _NMAIN, _D), jnp.float32)], axis=1
        )
        tout = pltpu.make_async_copy(
            vtout, out_hbm.at[pl.ds(_NMAIN, _NV - _NMAIN)], stail.at[1]
        )
        tout.start()
        out_copy(g - 1, lax.rem(g + 1, 2)).wait()
        out_copy(g, s).wait()
        tout.wait()


@jax.jit
def _tc_repack(tt, tail):
    return pl.pallas_call(
        _repack_body,
        grid=(_TC_GRID,),
        in_specs=[
            pl.BlockSpec(memory_space=pl.ANY),
            pl.BlockSpec(memory_space=pl.ANY),
        ],
        out_specs=pl.BlockSpec(memory_space=pl.ANY),
        out_shape=jax.ShapeDtypeStruct((_NV, _DP), jnp.float32),
        scratch_shapes=[
            pltpu.VMEM((2, _D, _TCW), jnp.float32),
            pltpu.VMEM((2, _TCW, _DP), jnp.float32),
            pltpu.VMEM((_NV - _NMAIN, _D), jnp.float32),
            pltpu.VMEM((_NV - _NMAIN, _DP), jnp.float32),
            pltpu.SemaphoreType.DMA((2,)),
            pltpu.SemaphoreType.DMA((2,)),
            pltpu.SemaphoreType.DMA((2,)),
        ],
    )(tt, tail)


def _gather_body(idx_hbm, table_hbm, out_hbm, idx_v, rows0, rows1, sem0, sem1):
    wid = lax.axis_index("s") * 2 + lax.axis_index("c")
    base = wid * _B_PER_W
    pltpu.sync_copy(idx_hbm.at[pl.ds(base, _B_PER_W)], idx_v)
    bufs = (rows0, rows1)
    sems = (sem0, sem1)
    copies = [None, None]
    copies[0] = pltpu.async_copy(
        table_hbm.at[idx_v.at[pl.ds(0, _CHUNK)]], bufs[0], sems[0]
    )
    for c in range(_NCHUNK):
        b = c % 2
        copies[b].wait()
        if c + 1 < _NCHUNK:
            nb = (c + 1) % 2
            copies[nb] = pltpu.async_copy(
                table_hbm.at[idx_v.at[pl.ds((c + 1) * _CHUNK, _CHUNK)]],
                bufs[nb],
                sems[nb],
            )
        pltpu.sync_copy(bufs[b], out_hbm.at[pl.ds(base + c * _CHUNK, _CHUNK)])


@jax.jit
def _sc_gather(idx_flat, table128):
    mesh = plsc.VectorSubcoreMesh(core_axis_name="c", subcore_axis_name="s")
    fn = functools.partial(
        pl.kernel,
        mesh=mesh,
        out_type=jax.ShapeDtypeStruct((_B, _DP), jnp.float32),
        scratch_types=[
            pltpu.VMEM((_B_PER_W,), jnp.int32),
            pltpu.VMEM((_CHUNK, _DP), jnp.float32),
            pltpu.VMEM((_CHUNK, _DP), jnp.float32),
            pltpu.SemaphoreType.DMA,
            pltpu.SemaphoreType.DMA,
        ],
        compiler_params=pltpu.CompilerParams(use_tc_tiling_on_sc=True),
    )(_gather_body)
    return fn(idx_flat, table128)


def kernel(inputs, entity_table, relation_table):
    idx_flat = inputs.reshape(_B).astype(jnp.int32)
    table128 = _tc_repack(entity_table.T, entity_table[_NMAIN:])
    out128 = _sc_gather(idx_flat, table128)
    return out128[:, :_D].reshape(_BATCH, _HIST, _D)
